# baseline (speedup 1.0000x reference)
import functools

import jax
import jax.numpy as jnp
from jax.experimental import pallas as pl
from jax.experimental.pallas import tpu as pltpu

M, K, N = 8192, 2048, 64
BLOCK_M = 512


def _gate_kernel(x_ref, wt_ref, o_ref):
    o_ref[...] = jax.lax.dot_general(
        x_ref[...].astype(jnp.bfloat16), wt_ref[...],
        dimension_numbers=(((1,), (0,)), ((), ())),
        preferred_element_type=jnp.float32,
    )


@functools.partial(jax.jit, static_argnames=())
def kernel(x, W):
    wt = W.T.astype(jnp.bfloat16)
    grid = (M // BLOCK_M,)
    return pl.pallas_call(
        _gate_kernel,
        grid=grid,
        in_specs=[
            pl.BlockSpec((BLOCK_M, K), lambda i: (i, 0)),
            pl.BlockSpec((K, N), lambda i: (0, 0)),
        ],
        out_specs=pl.BlockSpec((BLOCK_M, N), lambda i: (i, 0)),
        out_shape=jax.ShapeDtypeStruct((M, N), jnp.float32),
        compiler_params=pltpu.CompilerParams(
            dimension_semantics=("parallel",),
        ),
    )(x, wt)


# speedup vs baseline: 1.0973x; 1.0973x over previous
import jax
import jax.numpy as jnp
from jax.experimental import pallas as pl
from jax.experimental.pallas import tpu as pltpu

M, K, N = 8192, 2048, 64
BLOCK_M = 512


def _gate_kernel(xa_ref, xb_ref, wt_ref, o_ref):
    wt = wt_ref[...]
    o_ref[:BLOCK_M, :] = jax.lax.dot_general(
        xa_ref[...], wt, (((1,), (0,)), ((), ())),
        preferred_element_type=jnp.float32)
    o_ref[BLOCK_M:, :] = jax.lax.dot_general(
        xb_ref[...], wt, (((1,), (0,)), ((), ())),
        preferred_element_type=jnp.float32)


def kernel(x, W):
    wt = W.T
    grid = (M // (2 * BLOCK_M),)
    return pl.pallas_call(
        _gate_kernel,
        grid=grid,
        in_specs=[
            pl.BlockSpec((BLOCK_M, K), lambda i: (2 * i, 0)),
            pl.BlockSpec((BLOCK_M, K), lambda i: (2 * i + 1, 0)),
            pl.BlockSpec((K, N), lambda i: (0, 0)),
        ],
        out_specs=pl.BlockSpec((2 * BLOCK_M, N), lambda i: (i, 0)),
        out_shape=jax.ShapeDtypeStruct((M, N), jnp.float32),
        compiler_params=pltpu.CompilerParams(
            dimension_semantics=("arbitrary",),
        ),
    )(x, x, wt)
